# baseline (speedup 1.0000x reference)
import jax
import jax.numpy as jnp
from jax.experimental import pallas as pl
from jax.experimental.pallas import tpu as pltpu
from jax.experimental.pallas import tpu_sc as plsc

_BS = 512
_BR = 16
_LANES = 16
_Q = 1536


def _add_kernel(x_ref, pe_ref, o_ref):
    o_ref[...] = x_ref[...] + pe_ref[...][None, :, :]


def _tc_part(x, pe_table, n_rows):
    B, S, D = x.shape
    return pl.pallas_call(
        _add_kernel,
        grid=(n_rows // _BS,),
        in_specs=[
            pl.BlockSpec((B, _BS, D), lambda i: (0, i, 0)),
            pl.BlockSpec((_BS, D), lambda i: (i, 0)),
        ],
        out_specs=pl.BlockSpec((B, _BS, D), lambda i: (0, i, 0)),
        out_shape=jax.ShapeDtypeStruct((B, S, D), x.dtype),
        compiler_params=pltpu.CompilerParams(dimension_semantics=("parallel",)),
    )(x, pe_table)


def _sc_part(x, pe_table, q):
    B, S, D = x.shape
    x2 = x.reshape(B * S, D)
    sb_total = S // _BR
    qb = q // _BR
    base = (S - q) // _BR
    mesh = plsc.VectorSubcoreMesh(core_axis_name="core", subcore_axis_name="subcore")

    @pl.kernel(out_type=jax.ShapeDtypeStruct((B * q, D), x.dtype), mesh=mesh)
    def sc_kern(x_hbm, pe_hbm, o_hbm):
        def body(x_vmem, pe_vmem, o_vmem):
            @pl.loop(0, _BR)
            def _row(r):
                @pl.loop(0, D, step=_LANES)
                def _col(c):
                    slc = (pl.ds(r, 1), pl.ds(c, _LANES))
                    o_vmem.at[slc][...] = x_vmem.at[slc][...] + pe_vmem.at[slc][...]

        pltpu.emit_pipeline(
            body,
            grid=(qb, B),
            in_specs=[
                pl.BlockSpec((_BR, D), index_map=lambda i, b: (b * sb_total + base + i, 0)),
                pl.BlockSpec((_BR, D), index_map=lambda i, b: (base + i, 0)),
            ],
            out_specs=[pl.BlockSpec((_BR, D), index_map=lambda i, b: (b * qb + i, 0))],
            core_axis_name=("core", "subcore"),
            dimension_semantics=(pltpu.PARALLEL, pltpu.ARBITRARY),
        )(x_hbm, pe_hbm, o_hbm)

    return sc_kern(x2, pe_table).reshape(B, q, D)


def kernel(x, pe_table):
    B, S, D = x.shape
    tc_full = _tc_part(x, pe_table, S - _Q)
    sc_tail = _sc_part(x, pe_table, _Q)
    return jax.lax.dynamic_update_slice(tc_full, sc_tail, (0, S - _Q, 0))


# speedup vs baseline: 1.0013x; 1.0013x over previous
import jax
import jax.numpy as jnp
from jax.experimental import pallas as pl
from jax.experimental.pallas import tpu as pltpu
from jax.experimental.pallas import tpu_sc as plsc

_BR = 8
_LANES = 16


def kernel(x, pe_table):
    B, S, D = x.shape
    SB = S // _BR
    x2 = x.reshape(B * S, D)
    mesh = plsc.VectorSubcoreMesh(core_axis_name="core", subcore_axis_name="subcore")

    def _x_map(b):
        return lambda i: (b * SB + i, 0)

    @pl.kernel(out_type=jax.ShapeDtypeStruct((B * S, D), x.dtype), mesh=mesh)
    def sc_kern(x_hbm, pe_hbm, o_hbm):
        def body(*refs):
            x_refs = refs[:B]
            pe_vmem = refs[B]
            o_refs = refs[B + 1:]

            @pl.loop(0, _BR)
            def _row(r):
                @plsc.parallel_loop(0, D, step=_LANES, unroll=4)
                def _col(c):
                    slc = (pl.ds(r, 1), pl.ds(c, _LANES))
                    pv = pe_vmem.at[slc][...]
                    for xb, ob in zip(x_refs, o_refs):
                        ob.at[slc][...] = xb.at[slc][...] + pv

        pltpu.emit_pipeline(
            body,
            grid=(SB,),
            in_specs=[pl.BlockSpec((_BR, D), index_map=_x_map(b)) for b in range(B)]
            + [pl.BlockSpec((_BR, D), index_map=lambda i: (i, 0))],
            out_specs=[pl.BlockSpec((_BR, D), index_map=_x_map(b)) for b in range(B)],
            core_axis_name=("core", "subcore"),
            dimension_semantics=(pltpu.PARALLEL,),
        )(*([x_hbm] * B), pe_hbm, *([o_hbm] * B))

    return sc_kern(x2, pe_table).reshape(B, S, D)


# speedup vs baseline: 1.0036x; 1.0024x over previous
import jax
import jax.numpy as jnp
from jax.experimental import pallas as pl
from jax.experimental.pallas import tpu as pltpu
from jax.experimental.pallas import tpu_sc as plsc

_BR = 8
_LANES = 16


def kernel(x, pe_table):
    B, S, D = x.shape
    SB = S // _BR
    x2 = x.reshape(B * S, D)
    mesh = plsc.VectorSubcoreMesh(core_axis_name="core", subcore_axis_name="subcore")

    def _x_map(b):
        return lambda i: (b * SB + i, 0)

    @pl.kernel(out_type=jax.ShapeDtypeStruct((B * S, D), x.dtype), mesh=mesh)
    def sc_kern(x_hbm, pe_hbm, o_hbm):
        def body(*refs):
            x_refs = refs[:B]
            pe_vmem = refs[B]
            o_refs = refs[B + 1:]

            @pl.loop(0, _BR)
            def _row(r):
                @plsc.parallel_loop(0, D, step=_LANES, unroll=8)
                def _col(c):
                    slc = (pl.ds(r, 1), pl.ds(c, _LANES))
                    pv = pe_vmem.at[slc][...]
                    for xb, ob in zip(x_refs, o_refs):
                        ob.at[slc][...] = xb.at[slc][...] + pv

        pltpu.emit_pipeline(
            body,
            grid=(SB,),
            in_specs=[pl.BlockSpec((_BR, D), index_map=_x_map(b)) for b in range(B)]
            + [pl.BlockSpec((_BR, D), index_map=lambda i: (i, 0))],
            out_specs=[pl.BlockSpec((_BR, D), index_map=_x_map(b)) for b in range(B)],
            core_axis_name=("core", "subcore"),
            dimension_semantics=(pltpu.PARALLEL,),
        )(*([x_hbm] * B), pe_hbm, *([o_hbm] * B))

    return sc_kern(x2, pe_table).reshape(B, S, D)
